# baseline (speedup 1.0000x reference)
import functools

import jax
import jax.numpy as jnp
from jax import lax
from jax.experimental import pallas as pl
from jax.experimental.pallas import tpu as pltpu
from jax.experimental.pallas import tpu_sc as plsc

_NC = 2
_NS = 16
_NW = _NC * _NS
_L = 16
_BB = 128
_ET = 8


def _emb_body(seq, emb, ids_t, tok, pos, out,
              ids_v, pos_v, rows_v, dst_v,
              sem_g0, sem_g1, sem_w0, sem_w1):
    wid = lax.axis_index("s") * _NC + lax.axis_index("c")
    bb = wid
    b0 = pl.multiple_of(bb * _BB, 8)
    sem_g = (sem_g0, sem_g1)
    sem_w = (sem_w0, sem_w1)

    pltpu.sync_copy(ids_t.at[:, pl.ds(b0, _BB)], ids_v)
    pltpu.sync_copy(pos, pos_v)

    def start_gather(s, b):
        pltpu.async_copy(tok.at[ids_v.at[s]], rows_v.at[b], sem_g[b])

    def wait_gather(b):
        pltpu.make_async_copy(tok.at[ids_v.at[0]], rows_v.at[b],
                              sem_g[b]).wait()

    def start_wb(s, b):
        pltpu.async_copy(dst_v.at[b], out.at[s, :, bb], sem_w[b])

    def wait_wb(b):
        pltpu.make_async_copy(dst_v.at[b], out.at[0, :, bb], sem_w[b]).wait()

    row_idx = [lax.iota(jnp.int32, _L) + j * _L for j in range(_BB // _L)]

    def transpose_add(s, b):
        rows = rows_v.at[b]
        dst = dst_v.at[b]
        s_vec = jnp.full((_L,), s, jnp.int32)

        def tr(e, c):
            p = plsc.load_gather(pos_v, [s_vec, jnp.full((_L,), e, jnp.int32)])
            e_vec = jnp.full((_L,), e, jnp.int32)
            et = e // _ET
            ei = lax.rem(e, _ET)
            for j in range(_BB // _L):
                v = plsc.load_gather(rows, [row_idx[j], e_vec])
                dst[et, ei, pl.ds(j * _L, _L)] = v + p
            return c

        lax.fori_loop(0, emb, tr, 0)

    start_gather(0, 0)

    def step(t, carry):
        s0 = t * 2
        start_gather(s0 + 1, 1)
        wait_gather(0)

        @pl.when(t >= 1)
        def _():
            wait_wb(0)

        transpose_add(s0, 0)
        start_wb(s0, 0)

        @pl.when(t <= seq // 2 - 2)
        def _():
            start_gather(s0 + 2, 0)

        wait_gather(1)

        @pl.when(t >= 1)
        def _():
            wait_wb(1)

        transpose_add(s0 + 1, 1)
        start_wb(s0 + 1, 1)
        return carry

    lax.fori_loop(0, seq // 2, step, 0)
    wait_wb(0)
    wait_wb(1)


@functools.partial(jax.jit, static_argnames=("batch", "seq", "emb"))
def _emb_call(ids_t, token_table, pos_table, *, batch, seq, emb):
    mesh = plsc.VectorSubcoreMesh(core_axis_name="c", subcore_axis_name="s")
    kern = functools.partial(
        pl.kernel,
        out_type=jax.ShapeDtypeStruct(
            (seq, emb // _ET, batch // _BB, _ET, _BB), jnp.float32),
        mesh=mesh,
        scratch_types=[
            pltpu.VMEM((seq, _BB), jnp.int32),
            pltpu.VMEM((seq, emb), jnp.float32),
            pltpu.VMEM((2, _BB, emb), jnp.float32),
            pltpu.VMEM((2, emb // _ET, _ET, _BB), jnp.float32),
            pltpu.SemaphoreType.DMA,
            pltpu.SemaphoreType.DMA,
            pltpu.SemaphoreType.DMA,
            pltpu.SemaphoreType.DMA,
        ],
        compiler_params=pltpu.CompilerParams(use_tc_tiling_on_sc=False,
                                             needs_layout_passes=False),
    )(functools.partial(_emb_body, seq, emb))
    return kern(ids_t, token_table, pos_table)


def kernel(input_ids, token_table, pos_table):
    batch, seq = input_ids.shape
    emb = token_table.shape[1]
    ids_t = input_ids.astype(jnp.int32).T
    out5 = _emb_call(ids_t, token_table, pos_table,
                     batch=batch, seq=seq, emb=emb)
    return out5.transpose(2, 4, 0, 1, 3).reshape(batch, seq, emb)


# speedup vs baseline: 1.1324x; 1.1324x over previous
import functools

import jax
import jax.numpy as jnp
from jax import lax
from jax.experimental import pallas as pl
from jax.experimental.pallas import tpu as pltpu
from jax.experimental.pallas import tpu_sc as plsc

_NC = 2
_NS = 16
_NW = _NC * _NS
_L = 16
_BB = 128
_ET = 8


def _emb_body(seq, emb, ids_t, tok, pos, out,
              ids_v, pos_v, rows_v, dst_v,
              sem_g0, sem_g1, sem_w0, sem_w1):
    wid = lax.axis_index("s") * _NC + lax.axis_index("c")
    bb = wid
    b0 = pl.multiple_of(bb * _BB, 8)
    sem_g = (sem_g0, sem_g1)
    sem_w = (sem_w0, sem_w1)

    pltpu.sync_copy(ids_t.at[:, pl.ds(b0, _BB)], ids_v)
    pltpu.sync_copy(pos, pos_v)

    def start_gather(s, b):
        pltpu.async_copy(tok.at[ids_v.at[s]], rows_v.at[b], sem_g[b])

    def wait_gather(b):
        pltpu.make_async_copy(tok.at[ids_v.at[0]], rows_v.at[b],
                              sem_g[b]).wait()

    def start_wb(s, b):
        pltpu.async_copy(dst_v.at[b], out.at[s, :, bb], sem_w[b])

    def wait_wb(b):
        pltpu.make_async_copy(dst_v.at[b], out.at[0, :, bb], sem_w[b]).wait()

    iota = lax.iota(jnp.int32, _L)
    et_vec = [(iota + d * _L) // _ET for d in range(emb // _L)]
    ei_vec = [lax.rem(iota + d * _L, _ET) for d in range(emb // _L)]

    def transpose_add(s, b):
        rows = rows_v.at[b]
        dst = dst_v.at[b]
        pvec = [pos_v[s, pl.ds(d * _L, _L)] for d in range(emb // _L)]

        def tr(bl, c):
            b_vec = jnp.full((_L,), bl, jnp.int32)
            for d in range(emb // _L):
                v = rows[bl, pl.ds(d * _L, _L)] + pvec[d]
                plsc.store_scatter(dst, [et_vec[d], ei_vec[d], b_vec], v)
            return c

        lax.fori_loop(0, _BB, tr, 0, unroll=2)

    start_gather(0, 0)

    def step(t, carry):
        s0 = t * 2
        start_gather(s0 + 1, 1)
        wait_gather(0)

        @pl.when(t >= 1)
        def _():
            wait_wb(0)

        transpose_add(s0, 0)
        start_wb(s0, 0)

        @pl.when(t <= seq // 2 - 2)
        def _():
            start_gather(s0 + 2, 0)

        wait_gather(1)

        @pl.when(t >= 1)
        def _():
            wait_wb(1)

        transpose_add(s0 + 1, 1)
        start_wb(s0 + 1, 1)
        return carry

    lax.fori_loop(0, seq // 2, step, 0)
    wait_wb(0)
    wait_wb(1)


@functools.partial(jax.jit, static_argnames=("batch", "seq", "emb"))
def _emb_call(ids_t, token_table, pos_table, *, batch, seq, emb):
    mesh = plsc.VectorSubcoreMesh(core_axis_name="c", subcore_axis_name="s")
    kern = functools.partial(
        pl.kernel,
        out_type=jax.ShapeDtypeStruct(
            (seq, emb // _ET, batch // _BB, _ET, _BB), jnp.float32),
        mesh=mesh,
        scratch_types=[
            pltpu.VMEM((seq, _BB), jnp.int32),
            pltpu.VMEM((seq, emb), jnp.float32),
            pltpu.VMEM((2, _BB, emb), jnp.float32),
            pltpu.VMEM((2, emb // _ET, _ET, _BB), jnp.float32),
            pltpu.SemaphoreType.DMA,
            pltpu.SemaphoreType.DMA,
            pltpu.SemaphoreType.DMA,
            pltpu.SemaphoreType.DMA,
        ],
        compiler_params=pltpu.CompilerParams(use_tc_tiling_on_sc=False,
                                             needs_layout_passes=False),
    )(functools.partial(_emb_body, seq, emb))
    return kern(ids_t, token_table, pos_table)


def kernel(input_ids, token_table, pos_table):
    batch, seq = input_ids.shape
    emb = token_table.shape[1]
    ids_t = input_ids.astype(jnp.int32).T
    out5 = _emb_call(ids_t, token_table, pos_table,
                     batch=batch, seq=seq, emb=emb)
    return out5.transpose(2, 4, 0, 1, 3).reshape(batch, seq, emb)


# speedup vs baseline: 1.4949x; 1.3202x over previous
import functools

import jax
import jax.numpy as jnp
from jax import lax
from jax.experimental import pallas as pl
from jax.experimental.pallas import tpu as pltpu
from jax.experimental.pallas import tpu_sc as plsc

_NC = 2
_NS = 16
_NW = _NC * _NS
_L = 16
_BB = 128
_ET = 8


def _emb_body(seq, emb, ids_t, tok, pos, out,
              ids_v, pos_v, rows_v, dst_v,
              sem_g0, sem_g1, sem_w0, sem_w1):
    wid = lax.axis_index("s") * _NC + lax.axis_index("c")
    bb = wid
    b0 = pl.multiple_of(bb * _BB, 8)
    sem_g = (sem_g0, sem_g1)
    sem_w = (sem_w0, sem_w1)

    pltpu.sync_copy(ids_t.at[:, pl.ds(b0, _BB)], ids_v)
    pltpu.sync_copy(pos, pos_v)

    def start_gather(s, b):
        pltpu.async_copy(tok.at[ids_v.at[s]], rows_v.at[b], sem_g[b])

    def wait_gather(b):
        pltpu.make_async_copy(tok.at[ids_v.at[0]], rows_v.at[b],
                              sem_g[b]).wait()

    def start_wb(s, b):
        pltpu.async_copy(dst_v.at[b], out.at[s, :, bb], sem_w[b])

    def wait_wb(b):
        pltpu.make_async_copy(dst_v.at[b], out.at[0, :, bb], sem_w[b]).wait()

    iota = lax.iota(jnp.int32, _L)
    et_vec = [(iota + d * _L) // _ET for d in range(emb // _L)]
    ei_vec = [lax.rem(iota + d * _L, _ET) for d in range(emb // _L)]

    def transpose_add(s, b):
        rows = rows_v.at[b]
        dst = dst_v.at[b]
        pvec = [pos_v[s, pl.ds(d * _L, _L)] for d in range(emb // _L)]

        @plsc.parallel_loop(0, _BB, unroll=4)
        def _(bl):
            b_vec = jnp.full((_L,), bl, jnp.int32)
            for d in range(emb // _L):
                v = rows[bl, pl.ds(d * _L, _L)] + pvec[d]
                plsc.store_scatter(dst, [et_vec[d], ei_vec[d], b_vec], v)

    start_gather(0, 0)

    def step(t, carry):
        s0 = t * 2
        start_gather(s0 + 1, 1)
        wait_gather(0)

        @pl.when(t >= 1)
        def _():
            wait_wb(0)

        transpose_add(s0, 0)
        start_wb(s0, 0)

        @pl.when(t <= seq // 2 - 2)
        def _():
            start_gather(s0 + 2, 0)

        wait_gather(1)

        @pl.when(t >= 1)
        def _():
            wait_wb(1)

        transpose_add(s0 + 1, 1)
        start_wb(s0 + 1, 1)
        return carry

    lax.fori_loop(0, seq // 2, step, 0)
    wait_wb(0)
    wait_wb(1)


@functools.partial(jax.jit, static_argnames=("batch", "seq", "emb"))
def _emb_call(ids_t, token_table, pos_table, *, batch, seq, emb):
    mesh = plsc.VectorSubcoreMesh(core_axis_name="c", subcore_axis_name="s")
    kern = functools.partial(
        pl.kernel,
        out_type=jax.ShapeDtypeStruct(
            (seq, emb // _ET, batch // _BB, _ET, _BB), jnp.float32),
        mesh=mesh,
        scratch_types=[
            pltpu.VMEM((seq, _BB), jnp.int32),
            pltpu.VMEM((seq, emb), jnp.float32),
            pltpu.VMEM((2, _BB, emb), jnp.float32),
            pltpu.VMEM((2, emb // _ET, _ET, _BB), jnp.float32),
            pltpu.SemaphoreType.DMA,
            pltpu.SemaphoreType.DMA,
            pltpu.SemaphoreType.DMA,
            pltpu.SemaphoreType.DMA,
        ],
        compiler_params=pltpu.CompilerParams(use_tc_tiling_on_sc=False,
                                             needs_layout_passes=False),
    )(functools.partial(_emb_body, seq, emb))
    return kern(ids_t, token_table, pos_table)


def kernel(input_ids, token_table, pos_table):
    batch, seq = input_ids.shape
    emb = token_table.shape[1]
    ids_t = input_ids.astype(jnp.int32).T
    out5 = _emb_call(ids_t, token_table, pos_table,
                     batch=batch, seq=seq, emb=emb)
    return out5.transpose(2, 4, 0, 1, 3).reshape(batch, seq, emb)


# speedup vs baseline: 2.4625x; 1.6472x over previous
import functools

import jax
import jax.numpy as jnp
from jax import lax
from jax.experimental import pallas as pl
from jax.experimental.pallas import tpu as pltpu
from jax.experimental.pallas import tpu_sc as plsc

_NC = 2
_NS = 16
_NW = _NC * _NS
_L = 16
_BB = 128
_ET = 8


def _emb_body(seq, emb, ids_t, tok, pos, out,
              ids_v, pos_v, rows_v, dst_v,
              sem_g0, sem_g1, sem_w0, sem_w1):
    wid = lax.axis_index("s") * _NC + lax.axis_index("c")
    bb = wid
    b0 = pl.multiple_of(bb * _BB, 8)
    sem_g = (sem_g0, sem_g1)
    sem_w = (sem_w0, sem_w1)

    pltpu.sync_copy(ids_t.at[:, pl.ds(b0, _BB)], ids_v)
    pltpu.sync_copy(pos, pos_v)

    def start_gather(s, b):
        pltpu.async_copy(tok.at[ids_v.at[s]], rows_v.at[b], sem_g[b])

    def wait_gather(b):
        pltpu.make_async_copy(tok.at[ids_v.at[0]], rows_v.at[b],
                              sem_g[b]).wait()

    def start_wb(s, b):
        pltpu.async_copy(dst_v.at[b, :, :, pl.ds(0, _BB)], out.at[s, :, bb],
                         sem_w[b])

    def wait_wb(b):
        pltpu.make_async_copy(dst_v.at[b, :, :, pl.ds(0, _BB)],
                              out.at[0, :, bb], sem_w[b]).wait()

    iota = lax.iota(jnp.int32, _L)
    et_vec = [(iota + d * _L) // _ET for d in range(emb // _L)]
    ei_vec = [lax.rem(iota + d * _L, _ET) for d in range(emb // _L)]

    def transpose_add(s, b):
        rows = rows_v.at[b]
        dst = dst_v.at[b]
        pvec = [pos_v[s, pl.ds(d * _L, _L)] for d in range(emb // _L)]

        @plsc.parallel_loop(0, _BB, unroll=4)
        def _(bl):
            b_vec = jnp.full((_L,), bl, jnp.int32)
            for d in range(emb // _L):
                v = rows[bl, pl.ds(d * _L, _L)] + pvec[d]
                plsc.store_scatter(dst, [et_vec[d], ei_vec[d], b_vec], v)

    start_gather(0, 0)

    def step(t, carry):
        s0 = t * 2
        start_gather(s0 + 1, 1)
        wait_gather(0)

        @pl.when(t >= 1)
        def _():
            wait_wb(0)

        transpose_add(s0, 0)
        start_wb(s0, 0)

        @pl.when(t <= seq // 2 - 2)
        def _():
            start_gather(s0 + 2, 0)

        wait_gather(1)

        @pl.when(t >= 1)
        def _():
            wait_wb(1)

        transpose_add(s0 + 1, 1)
        start_wb(s0 + 1, 1)
        return carry

    lax.fori_loop(0, seq // 2, step, 0)
    wait_wb(0)
    wait_wb(1)


@functools.partial(jax.jit, static_argnames=("batch", "seq", "emb"))
def _emb_call(ids_t, token_table, pos_table, *, batch, seq, emb):
    mesh = plsc.VectorSubcoreMesh(core_axis_name="c", subcore_axis_name="s")
    kern = functools.partial(
        pl.kernel,
        out_type=jax.ShapeDtypeStruct(
            (seq, emb // _ET, batch // _BB, _ET, _BB), jnp.float32),
        mesh=mesh,
        scratch_types=[
            pltpu.VMEM((seq, _BB), jnp.int32),
            pltpu.VMEM((seq, emb), jnp.float32),
            pltpu.VMEM((2, _BB, emb), jnp.float32),
            pltpu.VMEM((2, emb // _ET, _ET, _BB + 1), jnp.float32),
            pltpu.SemaphoreType.DMA,
            pltpu.SemaphoreType.DMA,
            pltpu.SemaphoreType.DMA,
            pltpu.SemaphoreType.DMA,
        ],
        compiler_params=pltpu.CompilerParams(use_tc_tiling_on_sc=False,
                                             needs_layout_passes=False),
    )(functools.partial(_emb_body, seq, emb))
    return kern(ids_t, token_table, pos_table)


def kernel(input_ids, token_table, pos_table):
    batch, seq = input_ids.shape
    emb = token_table.shape[1]
    ids_t = input_ids.astype(jnp.int32).T
    out5 = _emb_call(ids_t, token_table, pos_table,
                     batch=batch, seq=seq, emb=emb)
    return out5.transpose(2, 4, 0, 1, 3).reshape(batch, seq, emb)


# speedup vs baseline: 2.5745x; 1.0455x over previous
import functools

import jax
import jax.numpy as jnp
from jax import lax
from jax.experimental import pallas as pl
from jax.experimental.pallas import tpu as pltpu
from jax.experimental.pallas import tpu_sc as plsc

_NC = 2
_NS = 16
_NW = _NC * _NS
_L = 16
_BB = 128
_ET = 8


def _emb_body(seq, emb, ids_t, tok, pos, out,
              ids_v, pos_v, rows_v, dst_v,
              sem_g0, sem_g1, sem_g2, sem_g3, sem_w0, sem_w1):
    wid = lax.axis_index("s") * _NC + lax.axis_index("c")
    bb = wid
    b0 = pl.multiple_of(bb * _BB, 8)
    sem_g = (sem_g0, sem_g1, sem_g2, sem_g3)
    sem_w = (sem_w0, sem_w1)

    pltpu.sync_copy(ids_t.at[:, pl.ds(b0, _BB)], ids_v)
    pltpu.sync_copy(pos, pos_v)

    def start_gather(s, b):
        pltpu.async_copy(tok.at[ids_v.at[s]], rows_v.at[b], sem_g[b])

    def wait_gather(b):
        pltpu.make_async_copy(tok.at[ids_v.at[0]], rows_v.at[b],
                              sem_g[b]).wait()

    def start_wb(s, b):
        pltpu.async_copy(dst_v.at[b, :, :, pl.ds(0, _BB)], out.at[s, :, bb],
                         sem_w[b])

    def wait_wb(b):
        pltpu.make_async_copy(dst_v.at[b, :, :, pl.ds(0, _BB)],
                              out.at[0, :, bb], sem_w[b]).wait()

    iota = lax.iota(jnp.int32, _L)
    et_vec = [(iota + d * _L) // _ET for d in range(emb // _L)]
    ei_vec = [lax.rem(iota + d * _L, _ET) for d in range(emb // _L)]

    def transpose_add(s, rb, wb):
        rows = rows_v.at[rb]
        dst = dst_v.at[wb]
        pvec = [pos_v[s, pl.ds(d * _L, _L)] for d in range(emb // _L)]

        @plsc.parallel_loop(0, _BB, unroll=4)
        def _(bl):
            b_vec = jnp.full((_L,), bl, jnp.int32)
            for d in range(emb // _L):
                v = rows[bl, pl.ds(d * _L, _L)] + pvec[d]
                plsc.store_scatter(dst, [et_vec[d], ei_vec[d], b_vec], v)

    start_gather(0, 0)
    start_gather(1, 1)

    def step(t, carry):
        for k in range(4):
            s = t * 4 + k

            if k < 2:
                start_gather(s + 2, (k + 2) % 4)
            else:
                @pl.when(t <= seq // 4 - 2)
                def _():
                    start_gather(s + 2, (k + 2) % 4)

            wait_gather(k)

            if k < 2:
                @pl.when(t >= 1)
                def _():
                    wait_wb(k % 2)
            else:
                wait_wb(k % 2)

            transpose_add(s, k, k % 2)
            start_wb(s, k % 2)
        return carry

    lax.fori_loop(0, seq // 4, step, 0)
    wait_wb(0)
    wait_wb(1)


@functools.partial(jax.jit, static_argnames=("batch", "seq", "emb"))
def _emb_call(ids_t, token_table, pos_table, *, batch, seq, emb):
    mesh = plsc.VectorSubcoreMesh(core_axis_name="c", subcore_axis_name="s")
    kern = functools.partial(
        pl.kernel,
        out_type=jax.ShapeDtypeStruct(
            (seq, emb // _ET, batch // _BB, _ET, _BB), jnp.float32),
        mesh=mesh,
        scratch_types=[
            pltpu.VMEM((seq, _BB), jnp.int32),
            pltpu.VMEM((seq, emb), jnp.float32),
            pltpu.VMEM((4, _BB, emb), jnp.float32),
            pltpu.VMEM((2, emb // _ET, _ET, _BB + 1), jnp.float32),
            pltpu.SemaphoreType.DMA,
            pltpu.SemaphoreType.DMA,
            pltpu.SemaphoreType.DMA,
            pltpu.SemaphoreType.DMA,
            pltpu.SemaphoreType.DMA,
            pltpu.SemaphoreType.DMA,
        ],
        compiler_params=pltpu.CompilerParams(use_tc_tiling_on_sc=False,
                                             needs_layout_passes=False),
    )(functools.partial(_emb_body, seq, emb))
    return kern(ids_t, token_table, pos_table)


def kernel(input_ids, token_table, pos_table):
    batch, seq = input_ids.shape
    emb = token_table.shape[1]
    ids_t = input_ids.astype(jnp.int32).T
    out5 = _emb_call(ids_t, token_table, pos_table,
                     batch=batch, seq=seq, emb=emb)
    return out5.transpose(2, 4, 0, 1, 3).reshape(batch, seq, emb)


# speedup vs baseline: 2.5747x; 1.0001x over previous
import functools

import jax
import jax.numpy as jnp
from jax import lax
from jax.experimental import pallas as pl
from jax.experimental.pallas import tpu as pltpu
from jax.experimental.pallas import tpu_sc as plsc

_NC = 2
_NS = 16
_NW = _NC * _NS
_L = 16
_BB = 128
_ET = 8


def _emb_body(seq, emb, ids_t, tok, pos, out,
              ids_v, pos_v, rows_v, dst_v,
              sem_g0, sem_g1, sem_g2, sem_g3, sem_g4, sem_g5, sem_g6, sem_g7,
              sem_w0, sem_w1):
    wid = lax.axis_index("s") * _NC + lax.axis_index("c")
    bb = wid
    b0 = pl.multiple_of(bb * _BB, 8)
    sem_g = (sem_g0, sem_g1, sem_g2, sem_g3, sem_g4, sem_g5, sem_g6, sem_g7)
    sem_w = (sem_w0, sem_w1)

    pltpu.sync_copy(ids_t.at[:, pl.ds(b0, _BB)], ids_v)
    pltpu.sync_copy(pos, pos_v)

    def start_gather(s, b):
        pltpu.async_copy(tok.at[ids_v.at[s]], rows_v.at[b], sem_g[b])

    def wait_gather(b):
        pltpu.make_async_copy(tok.at[ids_v.at[0]], rows_v.at[b],
                              sem_g[b]).wait()

    def start_wb(s, b):
        pltpu.async_copy(dst_v.at[b, :, :, pl.ds(0, _BB)], out.at[s, :, bb],
                         sem_w[b])

    def wait_wb(b):
        pltpu.make_async_copy(dst_v.at[b, :, :, pl.ds(0, _BB)],
                              out.at[0, :, bb], sem_w[b]).wait()

    iota = lax.iota(jnp.int32, _L)
    et_vec = [(iota + d * _L) // _ET for d in range(emb // _L)]
    ei_vec = [lax.rem(iota + d * _L, _ET) for d in range(emb // _L)]

    def transpose_add(s, rb, wb):
        rows = rows_v.at[rb]
        dst = dst_v.at[wb]
        pvec = [pos_v[s, pl.ds(d * _L, _L)] for d in range(emb // _L)]

        @plsc.parallel_loop(0, _BB, unroll=4)
        def _(bl):
            b_vec = jnp.full((_L,), bl, jnp.int32)
            for d in range(emb // _L):
                v = rows[bl, pl.ds(d * _L, _L)] + pvec[d]
                plsc.store_scatter(dst, [et_vec[d], ei_vec[d], b_vec], v)

    nb = 8
    ahead = 6
    for s in range(ahead):
        start_gather(s, s)

    def step(t, carry):
        for k in range(nb):
            s = t * nb + k

            if k < nb - ahead:
                start_gather(s + ahead, (k + ahead) % nb)
            else:
                @pl.when(t <= seq // nb - 2)
                def _():
                    start_gather(s + ahead, (k + ahead) % nb)

            wait_gather(k)

            if k < 2:
                @pl.when(t >= 1)
                def _():
                    wait_wb(k % 2)
            else:
                wait_wb(k % 2)

            transpose_add(s, k, k % 2)
            start_wb(s, k % 2)
        return carry

    lax.fori_loop(0, seq // nb, step, 0)
    wait_wb(0)
    wait_wb(1)


@functools.partial(jax.jit, static_argnames=("batch", "seq", "emb"))
def _emb_call(ids_t, token_table, pos_table, *, batch, seq, emb):
    mesh = plsc.VectorSubcoreMesh(core_axis_name="c", subcore_axis_name="s")
    kern = functools.partial(
        pl.kernel,
        out_type=jax.ShapeDtypeStruct(
            (seq, emb // _ET, batch // _BB, _ET, _BB), jnp.float32),
        mesh=mesh,
        scratch_types=[
            pltpu.VMEM((seq, _BB), jnp.int32),
            pltpu.VMEM((seq, emb), jnp.float32),
            pltpu.VMEM((8, _BB, emb), jnp.float32),
            pltpu.VMEM((2, emb // _ET, _ET, _BB + 1), jnp.float32),
        ] + [pltpu.SemaphoreType.DMA] * 10,
        compiler_params=pltpu.CompilerParams(use_tc_tiling_on_sc=False,
                                             needs_layout_passes=False),
    )(functools.partial(_emb_body, seq, emb))
    return kern(ids_t, token_table, pos_table)


def kernel(input_ids, token_table, pos_table):
    batch, seq = input_ids.shape
    emb = token_table.shape[1]
    ids_t = input_ids.astype(jnp.int32).T
    out5 = _emb_call(ids_t, token_table, pos_table,
                     batch=batch, seq=seq, emb=emb)
    return out5.transpose(2, 4, 0, 1, 3).reshape(batch, seq, emb)
